# baseline (speedup 1.0000x reference)
import jax
import jax.numpy as jnp
from jax.experimental import pallas as pl
from jax.experimental.pallas import tpu as pltpu


def _gcn_block_kernel(feat_ref, w_ref, adj_ref, out_ref, xw_ref):
    i = pl.program_id(0)

    @pl.when(i == 0)
    def _():
        xw_ref[...] = jnp.dot(feat_ref[...], w_ref[...],
                              preferred_element_type=jnp.float32
                              ).astype(jnp.bfloat16)

    acc = jnp.dot(adj_ref[...].astype(jnp.bfloat16), xw_ref[...],
                  preferred_element_type=jnp.float32)
    out_ref[...] = jnp.maximum(acc, 0.0)


def kernel(feat, adj, weight):
    n, d_in = feat.shape
    d_out = weight.shape[1]
    bm = 256
    return pl.pallas_call(
        _gcn_block_kernel,
        grid=(n // bm,),
        in_specs=[
            pl.BlockSpec((n, d_in), lambda i: (0, 0)),
            pl.BlockSpec((d_in, d_out), lambda i: (0, 0)),
            pl.BlockSpec((bm, n), lambda i: (i, 0)),
        ],
        out_specs=pl.BlockSpec((bm, d_out), lambda i: (i, 0)),
        out_shape=jax.ShapeDtypeStruct((n, d_out), jnp.float32),
        scratch_shapes=[pltpu.VMEM((n, d_out), jnp.bfloat16)],
    )(feat.astype(jnp.bfloat16), weight.astype(jnp.bfloat16), adj)


# speedup vs baseline: 1.0647x; 1.0647x over previous
import jax
import jax.numpy as jnp
from jax.experimental import pallas as pl
from jax.experimental.pallas import tpu as pltpu


def _gcn_block_kernel(feat_ref, w_ref, adj_ref, out_ref, xw_ref):
    i = pl.program_id(0)

    @pl.when(i == 0)
    def _():
        xw = jnp.dot(feat_ref[...], w_ref[...],
                     preferred_element_type=jnp.float32)
        xw_ref[...] = xw.astype(jnp.bfloat16)

    acc = jnp.dot(adj_ref[...].astype(jnp.bfloat16), xw_ref[...],
                  preferred_element_type=jnp.float32)
    out_ref[...] = jnp.maximum(acc, 0.0)


def kernel(feat, adj, weight):
    n, d_in = feat.shape
    d_out = weight.shape[1]
    bm = 256
    return pl.pallas_call(
        _gcn_block_kernel,
        grid=(n // bm,),
        in_specs=[
            pl.BlockSpec((n, d_in), lambda i: (0, 0)),
            pl.BlockSpec((d_in, d_out), lambda i: (0, 0)),
            pl.BlockSpec((bm, n), lambda i: (i, 0)),
        ],
        out_specs=pl.BlockSpec((bm, d_out), lambda i: (i, 0)),
        out_shape=jax.ShapeDtypeStruct((n, d_out), jnp.float32),
        scratch_shapes=[pltpu.VMEM((n, d_out), jnp.bfloat16)],
    )(feat, weight, adj)
